# baseline (speedup 1.0000x reference)
import functools

import jax
import jax.numpy as jnp
from jax import lax
from jax.experimental import pallas as pl
from jax.experimental.pallas import tpu as pltpu
from jax.experimental.pallas import tpu_sc as plsc

NUM_CORES = 2
NUM_SUBCORES = 16
LANES = 16
NW = NUM_CORES * NUM_SUBCORES
BATCH = 16384
EMBED_DIM = 32
N_PER_W = BATCH // NW
IDX_CHUNK = 128
N_CHUNKS = N_PER_W // IDX_CHUNK


def _mf_kernel(user_hbm, item_hbm, user_e_hbm, item_e_hbm, user_b_hbm,
               item_b_hbm, out_hbm, u_idx, i_idx, u_rows, i_rows, u_bias,
               i_bias, out_v, sem):
    wid = lax.axis_index("s") * NUM_CORES + lax.axis_index("c")
    base = wid * N_PER_W

    pltpu.sync_copy(user_hbm.at[pl.ds(wid * N_CHUNKS, N_CHUNKS)], u_idx)
    pltpu.sync_copy(item_hbm.at[pl.ds(wid * N_CHUNKS, N_CHUNKS)], i_idx)

    u_rows2d = u_rows
    i_rows2d = i_rows
    copies = []
    for j in range(N_CHUNKS):
        sl = pl.ds(j * IDX_CHUNK, IDX_CHUNK)
        copies.append(pltpu.async_copy(
            user_e_hbm.at[u_idx.at[j]], u_rows2d.at[sl], sem))
        copies.append(pltpu.async_copy(
            item_e_hbm.at[i_idx.at[j]], i_rows2d.at[sl], sem))
        copies.append(pltpu.async_copy(
            user_b_hbm.at[u_idx.at[j]], u_bias.at[sl], sem))
        copies.append(pltpu.async_copy(
            item_b_hbm.at[i_idx.at[j]], i_bias.at[sl], sem))
    for c in copies:
        c.wait()

    iota16 = lax.iota(jnp.int32, LANES)
    u_flat = u_rows
    i_flat = i_rows

    cols = [jnp.full((LANES,), d, dtype=jnp.int32) for d in range(EMBED_DIM)]

    def chunk_body(c, carry):
        row0 = c * LANES
        rows = row0 + iota16
        acc = u_bias[pl.ds(row0, LANES)] + i_bias[pl.ds(row0, LANES)]
        for d in range(EMBED_DIM):
            u = plsc.load_gather(u_flat, [rows, cols[d]])
            v = plsc.load_gather(i_flat, [rows, cols[d]])
            acc = acc + u * v
        out_v[pl.ds(row0, LANES)] = acc
        return carry

    lax.fori_loop(0, N_PER_W // LANES, chunk_body, 0)

    pltpu.sync_copy(out_v, out_hbm.at[pl.ds(base, N_PER_W)])


@jax.jit
def kernel(user, item, user_e, item_e, user_b, item_b):
    user2d = user.astype(jnp.int32).reshape(BATCH // IDX_CHUNK, IDX_CHUNK)
    item2d = item.astype(jnp.int32).reshape(BATCH // IDX_CHUNK, IDX_CHUNK)
    user_b1 = user_b.reshape(-1)
    item_b1 = item_b.reshape(-1)

    mesh = plsc.VectorSubcoreMesh(core_axis_name="c", subcore_axis_name="s")
    run = pl.kernel(
        _mf_kernel,
        out_type=jax.ShapeDtypeStruct((BATCH,), jnp.float32),
        mesh=mesh,
        scratch_types=[
            pltpu.VMEM((N_CHUNKS, IDX_CHUNK), jnp.int32),
            pltpu.VMEM((N_CHUNKS, IDX_CHUNK), jnp.int32),
            pltpu.VMEM((N_PER_W, EMBED_DIM), jnp.float32),
            pltpu.VMEM((N_PER_W, EMBED_DIM), jnp.float32),
            pltpu.VMEM((N_PER_W,), jnp.float32),
            pltpu.VMEM((N_PER_W,), jnp.float32),
            pltpu.VMEM((N_PER_W,), jnp.float32),
            pltpu.SemaphoreType.DMA,
        ],
        compiler_params=pltpu.CompilerParams(
            needs_layout_passes=False, use_tc_tiling_on_sc=False),
    )
    return run(user2d, item2d, user_e, item_e, user_b1, item_b1)
